# baseline (speedup 1.0000x reference)
import functools

import numpy as np

import jax
import jax.numpy as jnp
from jax import lax
from jax.experimental import pallas as pl
from jax.experimental.pallas import tpu as pltpu
from jax.experimental.pallas import tpu_sc as plsc

_NC, _NS = 2, 16
_NW = _NC * _NS
_G = 64


def _embed_prep(x, We, be2, Wa, Wb):
    N, D = x.shape
    H = We.shape[1]
    BN = 2000

    def body(x_ref, We_ref, be_ref, Wa_ref, Wb_ref, h_ref, a_ref, b_ref):
        h = jnp.dot(x_ref[...], We_ref[...], preferred_element_type=jnp.float32)
        h = h + be_ref[...]
        h_ref[...] = h
        a_ref[...] = jnp.dot(h, Wa_ref[...], preferred_element_type=jnp.float32)
        b_ref[...] = jnp.dot(h, Wb_ref[...], preferred_element_type=jnp.float32)

    blk = pl.BlockSpec((BN, D), lambda i: (i, 0))
    full = lambda s: pl.BlockSpec(s, lambda i: tuple(0 for _ in s))
    return pl.pallas_call(
        body,
        grid=(N // BN,),
        in_specs=[blk, full((D, H)), full((1, H)), full((H, H)), full((H, H))],
        out_specs=[pl.BlockSpec((BN, H), lambda i: (i, 0))] * 3,
        out_shape=[jax.ShapeDtypeStruct((N, H), jnp.float32)] * 3,
    )(x, We, be2, Wa, Wb)


def _update_prep(h, agg, Wa, Wb):
    N, H = h.shape
    BN = 2000

    def body(h_ref, agg_ref, Wa_ref, Wb_ref, h2_ref, a_ref, b_ref):
        h2 = jax.nn.softplus(h_ref[...] + agg_ref[0] + agg_ref[1])
        h2_ref[...] = h2
        a_ref[...] = jnp.dot(h2, Wa_ref[...], preferred_element_type=jnp.float32)
        b_ref[...] = jnp.dot(h2, Wb_ref[...], preferred_element_type=jnp.float32)

    full = lambda s: pl.BlockSpec(s, lambda i: tuple(0 for _ in s))
    return pl.pallas_call(
        body,
        grid=(N // BN,),
        in_specs=[
            pl.BlockSpec((BN, H), lambda i: (i, 0)),
            pl.BlockSpec((_NC, BN, H), lambda i: (0, i, 0)),
            full((H, H)),
            full((H, H)),
        ],
        out_specs=[pl.BlockSpec((BN, H), lambda i: (i, 0))] * 3,
        out_shape=[jax.ShapeDtypeStruct((N, H), jnp.float32)] * 3,
    )(h, agg, Wa, Wb)


def _edge_mlp(s, ea, W1e, b1, W2, b2):
    E, H = s.shape
    De = ea.shape[1]
    BE = 2560

    def body(s_ref, ea_ref, W1e_ref, b1_ref, W2_ref, b2_ref, msg_ref):
        e1 = s_ref[...].astype(jnp.float32) + jnp.dot(
            ea_ref[...], W1e_ref[...], preferred_element_type=jnp.float32)
        e1 = jnp.maximum(e1 + b1_ref[...], 0.0)
        e2 = jnp.dot(e1.astype(jnp.bfloat16),
                     W2_ref[...].astype(jnp.bfloat16),
                     preferred_element_type=jnp.float32)
        e2 = e2 + b2_ref[...]
        f = e2[:, :H]
        c = e2[:, H:]
        msg_ref[...] = jax.nn.sigmoid(f) * jax.nn.softplus(c)

    full = lambda sh: pl.BlockSpec(sh, lambda i: tuple(0 for _ in sh))
    return pl.pallas_call(
        body,
        grid=(E // BE,),
        in_specs=[
            pl.BlockSpec((BE, H), lambda i: (i, 0)),
            pl.BlockSpec((BE, De), lambda i: (i, 0)),
            full((De, H)),
            full((1, H)),
            full((H, 2 * H)),
            full((1, 2 * H)),
        ],
        out_specs=pl.BlockSpec((BE, H), lambda i: (i, 0)),
        out_shape=jax.ShapeDtypeStruct((E, H), jnp.float32),
    )(s, ea, W1e, b1, W2, b2)


def _pool_head(h, agg, bat_r, bat_c, Wm1, bm1, Wm2, bm2):
    N, H = h.shape
    G = _G

    def body(h_ref, agg_ref, br_ref, bc_ref, Wm1_ref, bm1_ref, Wm2_ref,
             bm2_ref, out_ref, mx_ref):
        hp = jax.nn.softplus(h_ref[...] + agg_ref[0] + agg_ref[1])
        br = br_ref[...]
        bc = bc_ref[...]
        gcol = lax.broadcasted_iota(jnp.int32, (G, 1), 0)
        onehot_t = (gcol == bc).astype(jnp.float32)
        sums = jnp.dot(onehot_t, hp, preferred_element_type=jnp.float32)
        counts = jnp.sum(onehot_t, axis=1, keepdims=True)
        mean = sums / jnp.maximum(counts, 1.0)
        neg = jnp.float32(-jnp.inf)

        def gbody(g, carry):
            m = br == g
            col = jnp.max(jnp.where(m, hp, neg), axis=0)
            mx_ref[pl.ds(g, 1), :] = col[None, :]
            return carry

        lax.fori_loop(0, G, gbody, 0)
        mxs = mx_ref[...]
        pooled = jnp.concatenate([mean, mxs], axis=1)
        hid = jnp.dot(pooled, Wm1_ref[...], preferred_element_type=jnp.float32)
        hid = jnp.maximum(hid + bm1_ref[...], 0.0)
        out = jnp.dot(hid, Wm2_ref[...], preferred_element_type=jnp.float32)
        out_ref[...] = out + bm2_ref[...]

    return pl.pallas_call(
        body,
        out_shape=jax.ShapeDtypeStruct((G, 1), jnp.float32),
        scratch_shapes=[pltpu.VMEM((G, H), jnp.float32)],
    )(h, agg, bat_r, bat_c, Wm1, bm1, Wm2, bm2)


@functools.lru_cache(maxsize=None)
def _build_gather(N, H, E, K):
    EW = E // _NW
    NCH = EW // K
    W = H // 2
    mesh = plsc.VectorSubcoreMesh(
        core_axis_name="c", subcore_axis_name="s",
        num_cores=_NC, num_subcores=_NS)

    @functools.partial(
        pl.kernel,
        out_type=jax.ShapeDtypeStruct((E * W,), jnp.int32),
        mesh=mesh,
        compiler_params=pltpu.CompilerParams(needs_layout_passes=False),
        scratch_types=[
            pltpu.VMEM((NCH, K), jnp.int32),
            pltpu.VMEM((NCH, K), jnp.int32),
            [pltpu.VMEM((K, H), jnp.float32)] * 2,
            [pltpu.VMEM((K, H), jnp.float32)] * 2,
            [pltpu.VMEM((K * W,), jnp.int32)] * 2,
            [pltpu.SemaphoreType.DMA] * 2,
            [pltpu.SemaphoreType.DMA] * 2,
            [pltpu.SemaphoreType.DMA] * 2,
        ],
    )
    def gather_sum(a_hbm, b_hbm, src3_hbm, dst3_hbm, s_hbm,
                   idxs, idxd, rows_a, rows_b, rows_o, sem_a, sem_b, sem_o):
        wid = lax.axis_index("s") * _NC + lax.axis_index("c")
        base = wid * EW
        pltpu.sync_copy(src3_hbm.at[wid], idxs)
        pltpu.sync_copy(dst3_hbm.at[wid], idxd)

        def start_gather(c, p):
            pltpu.async_copy(a_hbm.at[idxs.at[c]], rows_a[p], sem_a[p])
            pltpu.async_copy(b_hbm.at[idxd.at[c]], rows_b[p], sem_b[p])

        start_gather(0, 0)
        start_gather(1, 1)

        def finish(c, p):
            @pl.when(c >= 2)
            def _wait_out():
                pltpu.make_async_copy(
                    rows_o[p], s_hbm.at[pl.ds(0, K * W)], sem_o[p]).wait()

            pltpu.make_async_copy(
                a_hbm.at[pl.ds(0, K)], rows_a[p], sem_a[p]).wait()
            pltpu.make_async_copy(
                b_hbm.at[pl.ds(0, K)], rows_b[p], sem_b[p]).wait()

            def addrow(r, c2):
                for g in range(H // 32):
                    lo = pl.ds(g * 32, 16)
                    hi = pl.ds(g * 32 + 16, 16)
                    vlo = rows_a[p][r, lo] + rows_b[p][r, lo]
                    vhi = rows_a[p][r, hi] + rows_b[p][r, hi]
                    packed = plsc.pack(
                        vlo, vhi, format=plsc.PackFormat.INTERLEAVED)
                    rows_o[p][pl.ds(r * W + g * 16, 16)] = plsc.bitcast(
                        packed, jnp.int32)
                return c2

            lax.fori_loop(0, K, addrow, 0)
            pltpu.async_copy(
                rows_o[p], s_hbm.at[pl.ds((base + c * K) * W, K * W)],
                sem_o[p])

            @pl.when(c + 2 < NCH)
            def _next():
                start_gather(c + 2, p)

        def body(t, carry):
            c0 = 2 * t
            finish(c0, 0)

            @pl.when(c0 + 1 < NCH)
            def _odd():
                finish(c0 + 1, 1)

            return carry

        lax.fori_loop(0, (NCH + 1) // 2, body, 0)
        pltpu.make_async_copy(
            rows_o[0], s_hbm.at[pl.ds(0, K * W)], sem_o[0]).wait()
        pltpu.make_async_copy(
            rows_o[1], s_hbm.at[pl.ds(0, K * W)], sem_o[1]).wait()

    return gather_sum


@functools.lru_cache(maxsize=None)
def _build_scatter(N, H, E, K, ZB):
    EW = E // _NW
    RP = (N // _NS) // 8 * 8
    TAIL = N - _NS * RP
    mesh = plsc.VectorSubcoreMesh(
        core_axis_name="c", subcore_axis_name="s",
        num_cores=_NC, num_subcores=_NS)

    NCH = EW // K

    @functools.partial(
        pl.kernel,
        out_type=jax.ShapeDtypeStruct((_NC, N, H), jnp.float32),
        mesh=mesh,
        scratch_types=[
            pltpu.VMEM((NCH, K), jnp.int32),
            [pltpu.VMEM((K, H), jnp.float32)] * 2,
            pltpu.VMEM((ZB, H), jnp.float32),
            pltpu.VMEM_SHARED((N, H), jnp.float32),
            [pltpu.SemaphoreType.DMA] * 2,
        ],
    )
    def scatter_add(msg_hbm, dst3_hbm, agg_hbm, idxd, rows, zbuf, acc, sem):
        cid = lax.axis_index("c")
        sid = lax.axis_index("s")
        wid = sid * _NC + cid
        base = wid * EW
        pltpu.sync_copy(dst3_hbm.at[wid], idxd)
        for r in range(ZB):
            for cc in range(H // 16):
                zbuf[r, pl.ds(cc * 16, 16)] = jnp.zeros((16,), jnp.float32)
        for t in range(RP // ZB):
            pltpu.sync_copy(zbuf, acc.at[pl.ds(sid * RP + t * ZB, ZB)])

        @pl.when(sid == _NS - 1)
        def _zero_tail():
            for t in range(TAIL // ZB):
                pltpu.sync_copy(zbuf, acc.at[pl.ds(_NS * RP + t * ZB, ZB)])

        plsc.subcore_barrier()

        def start_load(c, p):
            pltpu.async_copy(msg_hbm.at[pl.ds(base + c * K, K)],
                             rows[p], sem[p])

        start_load(0, 0)
        start_load(1, 1)

        def do_chunk(c, p):
            pltpu.make_async_copy(
                msg_hbm.at[pl.ds(base, K)], rows[p], sem[p]).wait()
            pltpu.sync_copy(rows[p], acc.at[idxd.at[c]], add=True)

            @pl.when(c + 2 < NCH)
            def _next():
                start_load(c + 2, p)

        def body(t, carry):
            c0 = 2 * t
            do_chunk(c0, 0)

            @pl.when(c0 + 1 < NCH)
            def _odd():
                do_chunk(c0 + 1, 1)

            return carry

        lax.fori_loop(0, (NCH + 1) // 2, body, 0)
        plsc.subcore_barrier()
        pltpu.sync_copy(acc.at[pl.ds(sid * RP, RP)],
                        agg_hbm.at[cid, pl.ds(sid * RP, RP)])

        @pl.when(sid == _NS - 1)
        def _write_tail():
            pltpu.sync_copy(acc.at[pl.ds(_NS * RP, TAIL)],
                            agg_hbm.at[cid, pl.ds(_NS * RP, TAIL)])

    return scatter_add


def kernel(x, edge_index, edge_attr, batch, W_embed, b_embed,
           conv0_W1, conv0_b1, conv0_W2, conv0_b2,
           conv1_W1, conv1_b1, conv1_W2, conv1_b2,
           conv2_W1, conv2_b1, conv2_W2, conv2_b2,
           W_mlp1, b_mlp1, W_mlp2, b_mlp2):
    N, D = x.shape
    H = W_embed.shape[1]
    E = edge_index.shape[1]

    K = 80
    NCH = (E // _NW) // K
    src3 = edge_index[0].reshape(_NW, NCH, K)
    dst3 = edge_index[1].reshape(_NW, NCH, K)
    convs = [(conv0_W1, conv0_b1, conv0_W2, conv0_b2),
             (conv1_W1, conv1_b1, conv1_W2, conv1_b2),
             (conv2_W1, conv2_b1, conv2_W2, conv2_b2)]

    gather = _build_gather(N, H, E, K)
    scatter = _build_scatter(N, H, E, K, 16)

    perm = np.empty(H, dtype=np.int32)
    for g in range(H // 32):
        for j in range(16):
            perm[32 * g + 2 * j] = 32 * g + j
            perm[32 * g + 2 * j + 1] = 32 * g + 16 + j
    perm = jnp.asarray(perm)

    W1_0 = convs[0][0]
    h, a, b = _embed_prep(x, W_embed, b_embed.reshape(1, H),
                          W1_0[:H], W1_0[H:2 * H])

    out = None
    for l in range(3):
        W1, b1, W2, b2 = convs[l]
        sflat = gather(a, b, src3, dst3)
        s = lax.bitcast_convert_type(
            sflat.reshape(E, H // 2), jnp.bfloat16).reshape(E, H)
        msg = _edge_mlp(s, edge_attr, W1[2 * H:][:, perm],
                        b1[perm].reshape(1, -1), W2[perm, :],
                        b2.reshape(1, -1))
        agg = scatter(msg, dst3)
        if l < 2:
            W1n = convs[l + 1][0]
            h, a, b = _update_prep(h, agg, W1n[:H], W1n[H:2 * H])
        else:
            out = _pool_head(h, agg, batch.reshape(N, 1), batch.reshape(1, N),
                             W_mlp1, b_mlp1.reshape(1, -1),
                             W_mlp2, b_mlp2.reshape(1, 1))
    return out


# speedup vs baseline: 2.4702x; 2.4702x over previous
import functools

import numpy as np

import jax
import jax.numpy as jnp
from jax import lax
from jax.experimental import pallas as pl
from jax.experimental.pallas import tpu as pltpu
from jax.experimental.pallas import tpu_sc as plsc

_NC, _NS = 2, 16
_NW = _NC * _NS
_G = 64


def _embed_prep(x, We, be2, Wa, Wb):
    N, D = x.shape
    H = We.shape[1]
    BN = 2000

    def body(x_ref, We_ref, be_ref, Wa_ref, Wb_ref, h_ref, a_ref, b_ref):
        h = jnp.dot(x_ref[...], We_ref[...], preferred_element_type=jnp.float32)
        h = h + be_ref[...]
        h_ref[...] = h
        a_ref[...] = jnp.dot(h, Wa_ref[...], preferred_element_type=jnp.float32)
        b_ref[...] = jnp.dot(h, Wb_ref[...], preferred_element_type=jnp.float32)

    blk = pl.BlockSpec((BN, D), lambda i: (i, 0))
    full = lambda s: pl.BlockSpec(s, lambda i: tuple(0 for _ in s))
    return pl.pallas_call(
        body,
        grid=(N // BN,),
        in_specs=[blk, full((D, H)), full((1, H)), full((H, H)), full((H, H))],
        out_specs=[pl.BlockSpec((BN, H), lambda i: (i, 0))] * 3,
        out_shape=[jax.ShapeDtypeStruct((N, H), jnp.float32)] * 3,
    )(x, We, be2, Wa, Wb)


def _update_prep(h, agg, Wa, Wb):
    N, H = h.shape
    BN = 2000

    def body(h_ref, agg_ref, Wa_ref, Wb_ref, h2_ref, a_ref, b_ref):
        h2 = jax.nn.softplus(h_ref[...] + agg_ref[0] + agg_ref[1])
        h2_ref[...] = h2
        a_ref[...] = jnp.dot(h2, Wa_ref[...], preferred_element_type=jnp.float32)
        b_ref[...] = jnp.dot(h2, Wb_ref[...], preferred_element_type=jnp.float32)

    full = lambda s: pl.BlockSpec(s, lambda i: tuple(0 for _ in s))
    return pl.pallas_call(
        body,
        grid=(N // BN,),
        in_specs=[
            pl.BlockSpec((BN, H), lambda i: (i, 0)),
            pl.BlockSpec((_NC, BN, H), lambda i: (0, i, 0)),
            full((H, H)),
            full((H, H)),
        ],
        out_specs=[pl.BlockSpec((BN, H), lambda i: (i, 0))] * 3,
        out_shape=[jax.ShapeDtypeStruct((N, H), jnp.float32)] * 3,
    )(h, agg, Wa, Wb)


def _edge_mlp(s, ea, W1e, b1, W2, b2):
    E, H = s.shape
    De = ea.shape[1]
    BE = 2560

    def body(s_ref, ea_ref, W1e_ref, b1_ref, W2_ref, b2_ref, msg_ref):
        e1 = s_ref[...].astype(jnp.float32) + jnp.dot(
            ea_ref[...], W1e_ref[...], preferred_element_type=jnp.float32)
        e1 = jnp.maximum(e1 + b1_ref[...], 0.0)
        e2 = jnp.dot(e1.astype(jnp.bfloat16),
                     W2_ref[...].astype(jnp.bfloat16),
                     preferred_element_type=jnp.float32)
        e2 = e2 + b2_ref[...]
        f = e2[:, :H]
        c = e2[:, H:]
        msg_ref[...] = jax.nn.sigmoid(f) * jax.nn.softplus(c)

    full = lambda sh: pl.BlockSpec(sh, lambda i: tuple(0 for _ in sh))
    return pl.pallas_call(
        body,
        grid=(E // BE,),
        in_specs=[
            pl.BlockSpec((BE, H), lambda i: (i, 0)),
            pl.BlockSpec((BE, De), lambda i: (i, 0)),
            full((De, H)),
            full((1, H)),
            full((H, 2 * H)),
            full((1, 2 * H)),
        ],
        out_specs=pl.BlockSpec((BE, H), lambda i: (i, 0)),
        out_shape=jax.ShapeDtypeStruct((E, H), jnp.float32),
    )(s, ea, W1e, b1, W2, b2)


def _pool_head(h, agg, bat_r, bat_c, Wm1, bm1, Wm2, bm2):
    N, H = h.shape
    G = _G

    def body(h_ref, agg_ref, br_ref, bc_ref, Wm1_ref, bm1_ref, Wm2_ref,
             bm2_ref, out_ref, mx_ref):
        hp = jax.nn.softplus(h_ref[...] + agg_ref[0] + agg_ref[1])
        br = br_ref[...]
        bc = bc_ref[...]
        gcol = lax.broadcasted_iota(jnp.int32, (G, 1), 0)
        onehot_t = (gcol == bc).astype(jnp.float32)
        sums = jnp.dot(onehot_t, hp, preferred_element_type=jnp.float32)
        counts = jnp.sum(onehot_t, axis=1, keepdims=True)
        mean = sums / jnp.maximum(counts, 1.0)
        neg = jnp.float32(-jnp.inf)

        def gbody(g, carry):
            m = br == g
            col = jnp.max(jnp.where(m, hp, neg), axis=0)
            mx_ref[pl.ds(g, 1), :] = col[None, :]
            return carry

        lax.fori_loop(0, G, gbody, 0)
        mxs = mx_ref[...]
        pooled = jnp.concatenate([mean, mxs], axis=1)
        hid = jnp.dot(pooled, Wm1_ref[...], preferred_element_type=jnp.float32)
        hid = jnp.maximum(hid + bm1_ref[...], 0.0)
        out = jnp.dot(hid, Wm2_ref[...], preferred_element_type=jnp.float32)
        out_ref[...] = out + bm2_ref[...]

    return pl.pallas_call(
        body,
        out_shape=jax.ShapeDtypeStruct((G, 1), jnp.float32),
        scratch_shapes=[pltpu.VMEM((G, H), jnp.float32)],
    )(h, agg, bat_r, bat_c, Wm1, bm1, Wm2, bm2)


@functools.lru_cache(maxsize=None)
def _build_gather(N, H, E, K):
    EW = E // _NW
    NCH = EW // K
    mesh = plsc.VectorSubcoreMesh(
        core_axis_name="c", subcore_axis_name="s",
        num_cores=_NC, num_subcores=_NS)

    @functools.partial(
        pl.kernel,
        out_type=jax.ShapeDtypeStruct((E, H), jnp.float32),
        mesh=mesh,
        scratch_types=[
            pltpu.VMEM((NCH, K), jnp.int32),
            pltpu.VMEM((NCH, K), jnp.int32),
            [pltpu.VMEM((K, H), jnp.float32)] * 2,
            [pltpu.VMEM((K, H), jnp.float32)] * 2,
            [pltpu.VMEM((K, H), jnp.float32)] * 2,
            [pltpu.SemaphoreType.DMA] * 2,
            [pltpu.SemaphoreType.DMA] * 2,
            [pltpu.SemaphoreType.DMA] * 2,
        ],
    )
    def gather_sum(a_hbm, b_hbm, src3_hbm, dst3_hbm, s_hbm,
                   idxs, idxd, rows_a, rows_b, rows_o, sem_a, sem_b, sem_o):
        wid = lax.axis_index("s") * _NC + lax.axis_index("c")
        base = wid * EW
        pltpu.sync_copy(src3_hbm.at[wid], idxs)
        pltpu.sync_copy(dst3_hbm.at[wid], idxd)

        def start_gather(c, p):
            pltpu.async_copy(a_hbm.at[idxs.at[c]], rows_a[p], sem_a[p])
            pltpu.async_copy(b_hbm.at[idxd.at[c]], rows_b[p], sem_b[p])

        start_gather(0, 0)
        start_gather(1, 1)

        def finish(c, p):
            @pl.when(c >= 2)
            def _wait_out():
                pltpu.make_async_copy(
                    rows_o[p], s_hbm.at[pl.ds(base, K)], sem_o[p]).wait()

            pltpu.make_async_copy(
                a_hbm.at[pl.ds(0, K)], rows_a[p], sem_a[p]).wait()
            pltpu.make_async_copy(
                b_hbm.at[pl.ds(0, K)], rows_b[p], sem_b[p]).wait()

            def addrow(r, c2):
                for cc in range(H // 16):
                    sl = pl.ds(cc * 16, 16)
                    rows_o[p][r, sl] = rows_a[p][r, sl] + rows_b[p][r, sl]
                return c2

            lax.fori_loop(0, K, addrow, 0)
            pltpu.async_copy(rows_o[p], s_hbm.at[pl.ds(base + c * K, K)],
                             sem_o[p])

            @pl.when(c + 2 < NCH)
            def _next():
                start_gather(c + 2, p)

        def body(t, carry):
            c0 = 2 * t
            finish(c0, 0)

            @pl.when(c0 + 1 < NCH)
            def _odd():
                finish(c0 + 1, 1)

            return carry

        lax.fori_loop(0, (NCH + 1) // 2, body, 0)
        pltpu.make_async_copy(
            rows_o[0], s_hbm.at[pl.ds(base, K)], sem_o[0]).wait()
        pltpu.make_async_copy(
            rows_o[1], s_hbm.at[pl.ds(base, K)], sem_o[1]).wait()

    return gather_sum


@functools.lru_cache(maxsize=None)
def _build_scatter(N, H, E, K, ZB):
    EW = E // _NW
    RP = (N // _NS) // 8 * 8
    TAIL = N - _NS * RP
    mesh = plsc.VectorSubcoreMesh(
        core_axis_name="c", subcore_axis_name="s",
        num_cores=_NC, num_subcores=_NS)

    NCH = EW // K

    @functools.partial(
        pl.kernel,
        out_type=jax.ShapeDtypeStruct((_NC, N, H), jnp.float32),
        mesh=mesh,
        scratch_types=[
            pltpu.VMEM((NCH, K), jnp.int32),
            [pltpu.VMEM((K, H), jnp.float32)] * 2,
            pltpu.VMEM((ZB, H), jnp.float32),
            pltpu.VMEM_SHARED((N, H), jnp.float32),
            [pltpu.SemaphoreType.DMA] * 2,
        ],
    )
    def scatter_add(msg_hbm, dst3_hbm, agg_hbm, idxd, rows, zbuf, acc, sem):
        cid = lax.axis_index("c")
        sid = lax.axis_index("s")
        wid = sid * _NC + cid
        base = wid * EW
        pltpu.sync_copy(dst3_hbm.at[wid], idxd)
        for r in range(ZB):
            for cc in range(H // 16):
                zbuf[r, pl.ds(cc * 16, 16)] = jnp.zeros((16,), jnp.float32)
        for t in range(RP // ZB):
            pltpu.sync_copy(zbuf, acc.at[pl.ds(sid * RP + t * ZB, ZB)])

        @pl.when(sid == _NS - 1)
        def _zero_tail():
            for t in range(TAIL // ZB):
                pltpu.sync_copy(zbuf, acc.at[pl.ds(_NS * RP + t * ZB, ZB)])

        plsc.subcore_barrier()

        def start_load(c, p):
            pltpu.async_copy(msg_hbm.at[pl.ds(base + c * K, K)],
                             rows[p], sem[p])

        start_load(0, 0)
        start_load(1, 1)

        def do_chunk(c, p):
            pltpu.make_async_copy(
                msg_hbm.at[pl.ds(base, K)], rows[p], sem[p]).wait()
            pltpu.sync_copy(rows[p], acc.at[idxd.at[c]], add=True)

            @pl.when(c + 2 < NCH)
            def _next():
                start_load(c + 2, p)

        def body(t, carry):
            c0 = 2 * t
            do_chunk(c0, 0)

            @pl.when(c0 + 1 < NCH)
            def _odd():
                do_chunk(c0 + 1, 1)

            return carry

        lax.fori_loop(0, (NCH + 1) // 2, body, 0)
        plsc.subcore_barrier()
        pltpu.sync_copy(acc.at[pl.ds(sid * RP, RP)],
                        agg_hbm.at[cid, pl.ds(sid * RP, RP)])

        @pl.when(sid == _NS - 1)
        def _write_tail():
            pltpu.sync_copy(acc.at[pl.ds(_NS * RP, TAIL)],
                            agg_hbm.at[cid, pl.ds(_NS * RP, TAIL)])

    return scatter_add


def kernel(x, edge_index, edge_attr, batch, W_embed, b_embed,
           conv0_W1, conv0_b1, conv0_W2, conv0_b2,
           conv1_W1, conv1_b1, conv1_W2, conv1_b2,
           conv2_W1, conv2_b1, conv2_W2, conv2_b2,
           W_mlp1, b_mlp1, W_mlp2, b_mlp2):
    N, D = x.shape
    H = W_embed.shape[1]
    E = edge_index.shape[1]

    K = 80
    NCH = (E // _NW) // K
    src3 = edge_index[0].reshape(_NW, NCH, K)
    dst3 = edge_index[1].reshape(_NW, NCH, K)
    convs = [(conv0_W1, conv0_b1, conv0_W2, conv0_b2),
             (conv1_W1, conv1_b1, conv1_W2, conv1_b2),
             (conv2_W1, conv2_b1, conv2_W2, conv2_b2)]

    gather = _build_gather(N, H, E, K)
    scatter = _build_scatter(N, H, E, K, 16)

    W1_0 = convs[0][0]
    h, a, b = _embed_prep(x, W_embed, b_embed.reshape(1, H),
                          W1_0[:H], W1_0[H:2 * H])

    out = None
    for l in range(3):
        W1, b1, W2, b2 = convs[l]
        s = gather(a, b, src3, dst3)
        msg = _edge_mlp(s, edge_attr, W1[2 * H:], b1.reshape(1, -1),
                        W2, b2.reshape(1, -1))
        agg = scatter(msg, dst3)
        if l < 2:
            W1n = convs[l + 1][0]
            h, a, b = _update_prep(h, agg, W1n[:H], W1n[H:2 * H])
        else:
            out = _pool_head(h, agg, batch.reshape(N, 1), batch.reshape(1, N),
                             W_mlp1, b_mlp1.reshape(1, -1),
                             W_mlp2, b_mlp2.reshape(1, 1))
    return out


# speedup vs baseline: 2.6404x; 1.0689x over previous
import functools

import numpy as np

import jax
import jax.numpy as jnp
from jax import lax
from jax.experimental import pallas as pl
from jax.experimental.pallas import tpu as pltpu
from jax.experimental.pallas import tpu_sc as plsc

_NC, _NS = 2, 16
_NW = _NC * _NS
_G = 64


def _embed_prep(x, We, be2, Wa, Wb):
    N, D = x.shape
    H = We.shape[1]
    BN = 2000

    def body(x_ref, We_ref, be_ref, Wa_ref, Wb_ref, h_ref, a_ref, b_ref):
        h = jnp.dot(x_ref[...], We_ref[...], preferred_element_type=jnp.float32)
        h = h + be_ref[...]
        h_ref[...] = h
        a_ref[...] = jnp.dot(h, Wa_ref[...], preferred_element_type=jnp.float32)
        b_ref[...] = jnp.dot(h, Wb_ref[...], preferred_element_type=jnp.float32)

    blk = pl.BlockSpec((BN, D), lambda i: (i, 0))
    full = lambda s: pl.BlockSpec(s, lambda i: tuple(0 for _ in s))
    return pl.pallas_call(
        body,
        grid=(N // BN,),
        in_specs=[blk, full((D, H)), full((1, H)), full((H, H)), full((H, H))],
        out_specs=[pl.BlockSpec((BN, H), lambda i: (i, 0))] * 3,
        out_shape=[jax.ShapeDtypeStruct((N, H), jnp.float32)] * 3,
    )(x, We, be2, Wa, Wb)


def _update_prep(h, aggA, aggB, Wa, Wb):
    N, H = h.shape
    BN = 2000

    def body(h_ref, aggA_ref, aggB_ref, Wa_ref, Wb_ref, h2_ref, a_ref, b_ref):
        agg = aggA_ref[0] + aggA_ref[1] + aggB_ref[0] + aggB_ref[1]
        h2 = jax.nn.softplus(h_ref[...] + agg)
        h2_ref[...] = h2
        a_ref[...] = jnp.dot(h2, Wa_ref[...], preferred_element_type=jnp.float32)
        b_ref[...] = jnp.dot(h2, Wb_ref[...], preferred_element_type=jnp.float32)

    full = lambda s: pl.BlockSpec(s, lambda i: tuple(0 for _ in s))
    return pl.pallas_call(
        body,
        grid=(N // BN,),
        in_specs=[
            pl.BlockSpec((BN, H), lambda i: (i, 0)),
            pl.BlockSpec((_NC, BN, H), lambda i: (0, i, 0)),
            pl.BlockSpec((_NC, BN, H), lambda i: (0, i, 0)),
            full((H, H)),
            full((H, H)),
        ],
        out_specs=[pl.BlockSpec((BN, H), lambda i: (i, 0))] * 3,
        out_shape=[jax.ShapeDtypeStruct((N, H), jnp.float32)] * 3,
    )(h, aggA, aggB, Wa, Wb)


def _edge_mlp(s, ea, W1e, b1, W2, b2):
    E, H = s.shape
    De = ea.shape[1]
    BE = 3200

    def body(s_ref, ea_ref, W1e_ref, b1_ref, W2_ref, b2_ref, msg_ref):
        e1 = s_ref[...] + jnp.dot(ea_ref[...], W1e_ref[...],
                                  preferred_element_type=jnp.float32)
        e1 = jnp.maximum(e1 + b1_ref[...], 0.0)
        e2 = jnp.dot(e1, W2_ref[...], preferred_element_type=jnp.float32)
        e2 = e2 + b2_ref[...]
        f = e2[:, :H]
        c = e2[:, H:]
        msg_ref[...] = jax.nn.sigmoid(f) * jax.nn.softplus(c)

    full = lambda sh: pl.BlockSpec(sh, lambda i: tuple(0 for _ in sh))
    return pl.pallas_call(
        body,
        grid=(E // BE,),
        in_specs=[
            pl.BlockSpec((BE, H), lambda i: (i, 0)),
            pl.BlockSpec((BE, De), lambda i: (i, 0)),
            full((De, H)),
            full((1, H)),
            full((H, 2 * H)),
            full((1, 2 * H)),
        ],
        out_specs=pl.BlockSpec((BE, H), lambda i: (i, 0)),
        out_shape=jax.ShapeDtypeStruct((E, H), jnp.float32),
    )(s, ea, W1e, b1, W2, b2)


def _pool_head(h, aggA, aggB, bat_r, bat_c, Wm1, bm1, Wm2, bm2):
    N, H = h.shape
    G = _G

    def body(h_ref, aggA_ref, aggB_ref, br_ref, bc_ref, Wm1_ref, bm1_ref,
             Wm2_ref, bm2_ref, out_ref, mx_ref):
        agg = aggA_ref[0] + aggA_ref[1] + aggB_ref[0] + aggB_ref[1]
        hp = jax.nn.softplus(h_ref[...] + agg)
        br = br_ref[...]
        bc = bc_ref[...]
        gcol = lax.broadcasted_iota(jnp.int32, (G, 1), 0)
        onehot_t = (gcol == bc).astype(jnp.float32)
        sums = jnp.dot(onehot_t, hp, preferred_element_type=jnp.float32)
        counts = jnp.sum(onehot_t, axis=1, keepdims=True)
        mean = sums / jnp.maximum(counts, 1.0)
        neg = jnp.float32(-jnp.inf)

        def gbody(g, carry):
            m = br == g
            col = jnp.max(jnp.where(m, hp, neg), axis=0)
            mx_ref[pl.ds(g, 1), :] = col[None, :]
            return carry

        lax.fori_loop(0, G, gbody, 0)
        mxs = mx_ref[...]
        pooled = jnp.concatenate([mean, mxs], axis=1)
        hid = jnp.dot(pooled, Wm1_ref[...], preferred_element_type=jnp.float32)
        hid = jnp.maximum(hid + bm1_ref[...], 0.0)
        out = jnp.dot(hid, Wm2_ref[...], preferred_element_type=jnp.float32)
        out_ref[...] = out + bm2_ref[...]

    return pl.pallas_call(
        body,
        out_shape=jax.ShapeDtypeStruct((G, 1), jnp.float32),
        scratch_shapes=[pltpu.VMEM((G, H), jnp.float32)],
    )(h, aggA, aggB, bat_r, bat_c, Wm1, bm1, Wm2, bm2)


@functools.lru_cache(maxsize=None)
def _build_gather(N, H, E, K):
    EW = E // _NW
    NCH = EW // K
    mesh = plsc.VectorSubcoreMesh(
        core_axis_name="c", subcore_axis_name="s",
        num_cores=_NC, num_subcores=_NS)

    @functools.partial(
        pl.kernel,
        out_type=jax.ShapeDtypeStruct((E, H), jnp.float32),
        mesh=mesh,
        scratch_types=[
            pltpu.VMEM((NCH, K), jnp.int32),
            pltpu.VMEM((NCH, K), jnp.int32),
            [pltpu.VMEM((K, H), jnp.float32)] * 2,
            [pltpu.VMEM((K, H), jnp.float32)] * 2,
            [pltpu.VMEM((K, H), jnp.float32)] * 2,
            [pltpu.SemaphoreType.DMA] * 2,
            [pltpu.SemaphoreType.DMA] * 2,
            [pltpu.SemaphoreType.DMA] * 2,
        ],
    )
    def gather_sum(a_hbm, b_hbm, src3_hbm, dst3_hbm, s_hbm,
                   idxs, idxd, rows_a, rows_b, rows_o, sem_a, sem_b, sem_o):
        wid = lax.axis_index("s") * _NC + lax.axis_index("c")
        base = wid * EW
        pltpu.sync_copy(src3_hbm.at[wid], idxs)
        pltpu.sync_copy(dst3_hbm.at[wid], idxd)

        def start_gather(c, p):
            pltpu.async_copy(a_hbm.at[idxs.at[c]], rows_a[p], sem_a[p])
            pltpu.async_copy(b_hbm.at[idxd.at[c]], rows_b[p], sem_b[p])

        start_gather(0, 0)
        start_gather(1, 1)

        def finish(c, p):
            @pl.when(c >= 2)
            def _wait_out():
                pltpu.make_async_copy(
                    rows_o[p], s_hbm.at[pl.ds(base, K)], sem_o[p]).wait()

            pltpu.make_async_copy(
                a_hbm.at[pl.ds(0, K)], rows_a[p], sem_a[p]).wait()
            pltpu.make_async_copy(
                b_hbm.at[pl.ds(0, K)], rows_b[p], sem_b[p]).wait()

            def addrow(r, c2):
                for cc in range(H // 16):
                    sl = pl.ds(cc * 16, 16)
                    rows_o[p][r, sl] = rows_a[p][r, sl] + rows_b[p][r, sl]
                return c2

            lax.fori_loop(0, K, addrow, 0)
            pltpu.async_copy(rows_o[p], s_hbm.at[pl.ds(base + c * K, K)],
                             sem_o[p])

            @pl.when(c + 2 < NCH)
            def _next():
                start_gather(c + 2, p)

        def body(t, carry):
            c0 = 2 * t
            finish(c0, 0)

            @pl.when(c0 + 1 < NCH)
            def _odd():
                finish(c0 + 1, 1)

            return carry

        lax.fori_loop(0, (NCH + 1) // 2, body, 0)
        pltpu.make_async_copy(
            rows_o[0], s_hbm.at[pl.ds(base, K)], sem_o[0]).wait()
        pltpu.make_async_copy(
            rows_o[1], s_hbm.at[pl.ds(base, K)], sem_o[1]).wait()

    return gather_sum


@functools.lru_cache(maxsize=None)
def _build_scatter(N, H, E, K, ZB):
    EW = E // _NW
    RP = (N // _NS) // 8 * 8
    TAIL = N - _NS * RP
    mesh = plsc.VectorSubcoreMesh(
        core_axis_name="c", subcore_axis_name="s",
        num_cores=_NC, num_subcores=_NS)

    NCH = EW // K

    @functools.partial(
        pl.kernel,
        out_type=jax.ShapeDtypeStruct((_NC, N, H), jnp.float32),
        mesh=mesh,
        scratch_types=[
            pltpu.VMEM((NCH, K), jnp.int32),
            [pltpu.VMEM((K, H), jnp.float32)] * 2,
            pltpu.VMEM((ZB, H), jnp.float32),
            pltpu.VMEM_SHARED((N, H), jnp.float32),
            [pltpu.SemaphoreType.DMA] * 2,
        ],
    )
    def scatter_add(msg_hbm, dst3_hbm, agg_hbm, idxd, rows, zbuf, acc, sem):
        cid = lax.axis_index("c")
        sid = lax.axis_index("s")
        wid = sid * _NC + cid
        base = wid * EW
        pltpu.sync_copy(dst3_hbm.at[wid], idxd)
        for r in range(ZB):
            for cc in range(H // 16):
                zbuf[r, pl.ds(cc * 16, 16)] = jnp.zeros((16,), jnp.float32)
        for t in range(RP // ZB):
            pltpu.sync_copy(zbuf, acc.at[pl.ds(sid * RP + t * ZB, ZB)])

        @pl.when(sid == _NS - 1)
        def _zero_tail():
            for t in range(TAIL // ZB):
                pltpu.sync_copy(zbuf, acc.at[pl.ds(_NS * RP + t * ZB, ZB)])

        plsc.subcore_barrier()

        def start_load(c, p):
            pltpu.async_copy(msg_hbm.at[pl.ds(base + c * K, K)],
                             rows[p], sem[p])

        start_load(0, 0)
        start_load(1, 1)

        def do_chunk(c, p):
            pltpu.make_async_copy(
                msg_hbm.at[pl.ds(base, K)], rows[p], sem[p]).wait()
            pltpu.sync_copy(rows[p], acc.at[idxd.at[c]], add=True)

            @pl.when(c + 2 < NCH)
            def _next():
                start_load(c + 2, p)

        def body(t, carry):
            c0 = 2 * t
            do_chunk(c0, 0)

            @pl.when(c0 + 1 < NCH)
            def _odd():
                do_chunk(c0 + 1, 1)

            return carry

        lax.fori_loop(0, (NCH + 1) // 2, body, 0)
        plsc.subcore_barrier()
        pltpu.sync_copy(acc.at[pl.ds(sid * RP, RP)],
                        agg_hbm.at[cid, pl.ds(sid * RP, RP)])

        @pl.when(sid == _NS - 1)
        def _write_tail():
            pltpu.sync_copy(acc.at[pl.ds(_NS * RP, TAIL)],
                            agg_hbm.at[cid, pl.ds(_NS * RP, TAIL)])

    return scatter_add


def kernel(x, edge_index, edge_attr, batch, W_embed, b_embed,
           conv0_W1, conv0_b1, conv0_W2, conv0_b2,
           conv1_W1, conv1_b1, conv1_W2, conv1_b2,
           conv2_W1, conv2_b1, conv2_W2, conv2_b2,
           W_mlp1, b_mlp1, W_mlp2, b_mlp2):
    N, D = x.shape
    H = W_embed.shape[1]
    E = edge_index.shape[1]

    EH_ = E // 2
    K = 40
    NCH = (EH_ // _NW) // K
    srcA3 = edge_index[0][:EH_].reshape(_NW, NCH, K)
    dstA3 = edge_index[1][:EH_].reshape(_NW, NCH, K)
    srcB3 = edge_index[0][EH_:].reshape(_NW, NCH, K)
    dstB3 = edge_index[1][EH_:].reshape(_NW, NCH, K)
    eaA = edge_attr[:EH_]
    eaB = edge_attr[EH_:]
    convs = [(conv0_W1, conv0_b1, conv0_W2, conv0_b2),
             (conv1_W1, conv1_b1, conv1_W2, conv1_b2),
             (conv2_W1, conv2_b1, conv2_W2, conv2_b2)]

    gather = _build_gather(N, H, EH_, K)
    scatter = _build_scatter(N, H, EH_, K, 16)

    W1_0 = convs[0][0]
    h, a, b = _embed_prep(x, W_embed, b_embed.reshape(1, H),
                          W1_0[:H], W1_0[H:2 * H])

    out = None
    for l in range(3):
        W1, b1, W2, b2 = convs[l]
        sA = gather(a, b, srcA3, dstA3)
        sB = gather(a, b, srcB3, dstB3)
        msgA = _edge_mlp(sA, eaA, W1[2 * H:], b1.reshape(1, -1),
                         W2, b2.reshape(1, -1))
        msgB = _edge_mlp(sB, eaB, W1[2 * H:], b1.reshape(1, -1),
                         W2, b2.reshape(1, -1))
        aggA = scatter(msgA, dstA3)
        aggB = scatter(msgB, dstB3)
        if l < 2:
            W1n = convs[l + 1][0]
            h, a, b = _update_prep(h, aggA, aggB, W1n[:H], W1n[H:2 * H])
        else:
            out = _pool_head(h, aggA, aggB,
                             batch.reshape(N, 1), batch.reshape(1, N),
                             W_mlp1, b_mlp1.reshape(1, -1),
                             W_mlp2, b_mlp2.reshape(1, 1))
    return out


# speedup vs baseline: 2.7939x; 1.0582x over previous
import functools

import numpy as np

import jax
import jax.numpy as jnp
from jax import lax
from jax.experimental import pallas as pl
from jax.experimental.pallas import tpu as pltpu
from jax.experimental.pallas import tpu_sc as plsc

_NC, _NS = 2, 16
_NW = _NC * _NS
_G = 64


def _embed_prep(x, We, be2, Wa, Wb):
    N, D = x.shape
    H = We.shape[1]
    BN = 2000

    def body(x_ref, We_ref, be_ref, Wa_ref, Wb_ref, h_ref, a_ref, b_ref):
        h = jnp.dot(x_ref[...], We_ref[...], preferred_element_type=jnp.float32)
        h = h + be_ref[...]
        h_ref[...] = h
        a_ref[...] = jnp.dot(h, Wa_ref[...], preferred_element_type=jnp.float32)
        b_ref[...] = jnp.dot(h, Wb_ref[...], preferred_element_type=jnp.float32)

    blk = pl.BlockSpec((BN, D), lambda i: (i, 0))
    full = lambda s: pl.BlockSpec(s, lambda i: tuple(0 for _ in s))
    return pl.pallas_call(
        body,
        grid=(N // BN,),
        in_specs=[blk, full((D, H)), full((1, H)), full((H, H)), full((H, H))],
        out_specs=[pl.BlockSpec((BN, H), lambda i: (i, 0))] * 3,
        out_shape=[jax.ShapeDtypeStruct((N, H), jnp.float32)] * 3,
    )(x, We, be2, Wa, Wb)


def _update_prep(h, aggA, aggB, Wa, Wb):
    N, H = h.shape
    BN = 2000

    def body(h_ref, aggA_ref, aggB_ref, Wa_ref, Wb_ref, h2_ref, a_ref, b_ref):
        agg = aggA_ref[0] + aggA_ref[1] + aggB_ref[0] + aggB_ref[1]
        h2 = jax.nn.softplus(h_ref[...] + agg)
        h2_ref[...] = h2
        a_ref[...] = jnp.dot(h2, Wa_ref[...], preferred_element_type=jnp.float32)
        b_ref[...] = jnp.dot(h2, Wb_ref[...], preferred_element_type=jnp.float32)

    full = lambda s: pl.BlockSpec(s, lambda i: tuple(0 for _ in s))
    return pl.pallas_call(
        body,
        grid=(N // BN,),
        in_specs=[
            pl.BlockSpec((BN, H), lambda i: (i, 0)),
            pl.BlockSpec((_NC, BN, H), lambda i: (0, i, 0)),
            pl.BlockSpec((_NC, BN, H), lambda i: (0, i, 0)),
            full((H, H)),
            full((H, H)),
        ],
        out_specs=[pl.BlockSpec((BN, H), lambda i: (i, 0))] * 3,
        out_shape=[jax.ShapeDtypeStruct((N, H), jnp.float32)] * 3,
    )(h, aggA, aggB, Wa, Wb)


def _edge_mlp(s, ea, W1e, b1, W2, b2):
    E, H = s.shape
    De = ea.shape[1]
    BE = 3200

    def body(s_ref, ea_ref, W1e_ref, b1_ref, W2_ref, b2_ref, msg_ref):
        e1 = s_ref[...] + jnp.dot(ea_ref[...], W1e_ref[...],
                                  preferred_element_type=jnp.float32)
        e1 = jnp.maximum(e1 + b1_ref[...], 0.0)
        e2 = jnp.dot(e1, W2_ref[...], preferred_element_type=jnp.float32)
        e2 = e2 + b2_ref[...]
        f = e2[:, :H]
        c = e2[:, H:]
        msg_ref[...] = jax.nn.sigmoid(f) * jax.nn.softplus(c)

    full = lambda sh: pl.BlockSpec(sh, lambda i: tuple(0 for _ in sh))
    return pl.pallas_call(
        body,
        grid=(E // BE,),
        in_specs=[
            pl.BlockSpec((BE, H), lambda i: (i, 0)),
            pl.BlockSpec((BE, De), lambda i: (i, 0)),
            full((De, H)),
            full((1, H)),
            full((H, 2 * H)),
            full((1, 2 * H)),
        ],
        out_specs=pl.BlockSpec((BE, H), lambda i: (i, 0)),
        out_shape=jax.ShapeDtypeStruct((E, H), jnp.float32),
    )(s, ea, W1e, b1, W2, b2)


def _pool_head(h, aggA, aggB, bat_r, bat_c, Wm1, bm1, Wm2, bm2):
    N, H = h.shape
    G = _G

    def body(h_ref, aggA_ref, aggB_ref, br_ref, bc_ref, Wm1_ref, bm1_ref,
             Wm2_ref, bm2_ref, out_ref, mx_ref):
        agg = aggA_ref[0] + aggA_ref[1] + aggB_ref[0] + aggB_ref[1]
        hp = jax.nn.softplus(h_ref[...] + agg)
        br = br_ref[...]
        bc = bc_ref[...]
        gcol = lax.broadcasted_iota(jnp.int32, (G, 1), 0)
        onehot_t = (gcol == bc).astype(jnp.float32)
        sums = jnp.dot(onehot_t, hp, preferred_element_type=jnp.float32)
        counts = jnp.sum(onehot_t, axis=1, keepdims=True)
        mean = sums / jnp.maximum(counts, 1.0)
        neg = jnp.float32(-jnp.inf)

        def gbody(g, carry):
            m = br == g
            col = jnp.max(jnp.where(m, hp, neg), axis=0)
            mx_ref[pl.ds(g, 1), :] = col[None, :]
            return carry

        lax.fori_loop(0, G, gbody, 0)
        mxs = mx_ref[...]
        pooled = jnp.concatenate([mean, mxs], axis=1)
        hid = jnp.dot(pooled, Wm1_ref[...], preferred_element_type=jnp.float32)
        hid = jnp.maximum(hid + bm1_ref[...], 0.0)
        out = jnp.dot(hid, Wm2_ref[...], preferred_element_type=jnp.float32)
        out_ref[...] = out + bm2_ref[...]

    return pl.pallas_call(
        body,
        out_shape=jax.ShapeDtypeStruct((G, 1), jnp.float32),
        scratch_shapes=[pltpu.VMEM((G, H), jnp.float32)],
    )(h, aggA, aggB, bat_r, bat_c, Wm1, bm1, Wm2, bm2)


@functools.lru_cache(maxsize=None)
def _build_gather(N, H, E, K):
    EW = E // _NW
    NCH = EW // K
    mesh = plsc.VectorSubcoreMesh(
        core_axis_name="c", subcore_axis_name="s",
        num_cores=_NC, num_subcores=_NS)

    @functools.partial(
        pl.kernel,
        out_type=jax.ShapeDtypeStruct((E, H), jnp.float32),
        mesh=mesh,
        scratch_types=[
            pltpu.VMEM((NCH, K), jnp.int32),
            pltpu.VMEM((NCH, K), jnp.int32),
            [pltpu.VMEM((K, H), jnp.float32)] * 2,
            [pltpu.VMEM((K, H), jnp.float32)] * 2,
            [pltpu.VMEM((K, H), jnp.float32)] * 2,
            [pltpu.SemaphoreType.DMA] * 2,
            [pltpu.SemaphoreType.DMA] * 2,
            [pltpu.SemaphoreType.DMA] * 2,
        ],
    )
    def gather_sum(a_hbm, b_hbm, src3_hbm, dst3_hbm, s_hbm,
                   idxs, idxd, rows_a, rows_b, rows_o, sem_a, sem_b, sem_o):
        wid = lax.axis_index("s") * _NC + lax.axis_index("c")
        base = wid * EW
        pltpu.sync_copy(src3_hbm.at[wid], idxs)
        pltpu.sync_copy(dst3_hbm.at[wid], idxd)

        def start_gather(c, p):
            pltpu.async_copy(a_hbm.at[idxs.at[c]], rows_a[p], sem_a[p])
            pltpu.async_copy(b_hbm.at[idxd.at[c]], rows_b[p], sem_b[p])

        start_gather(0, 0)
        start_gather(1, 1)

        def finish(c, p):
            @pl.when(c >= 2)
            def _wait_out():
                pltpu.make_async_copy(
                    rows_o[p], s_hbm.at[pl.ds(base, K)], sem_o[p]).wait()

            pltpu.make_async_copy(
                a_hbm.at[pl.ds(0, K)], rows_a[p], sem_a[p]).wait()
            pltpu.make_async_copy(
                b_hbm.at[pl.ds(0, K)], rows_b[p], sem_b[p]).wait()

            def addrow(r, c2):
                for cc in range(H // 16):
                    sl = pl.ds(cc * 16, 16)
                    rows_o[p][r, sl] = rows_a[p][r, sl] + rows_b[p][r, sl]
                return c2

            lax.fori_loop(0, K, addrow, 0)
            pltpu.async_copy(rows_o[p], s_hbm.at[pl.ds(base + c * K, K)],
                             sem_o[p])

            @pl.when(c + 2 < NCH)
            def _next():
                start_gather(c + 2, p)

        def body(t, carry):
            c0 = 2 * t
            finish(c0, 0)

            @pl.when(c0 + 1 < NCH)
            def _odd():
                finish(c0 + 1, 1)

            return carry

        lax.fori_loop(0, (NCH + 1) // 2, body, 0)
        pltpu.make_async_copy(
            rows_o[0], s_hbm.at[pl.ds(base, K)], sem_o[0]).wait()
        pltpu.make_async_copy(
            rows_o[1], s_hbm.at[pl.ds(base, K)], sem_o[1]).wait()

    return gather_sum


@functools.lru_cache(maxsize=None)
def _build_scatter(N, H, E, K, ZB):
    EW = E // _NW
    RP = (N // _NS) // 8 * 8
    TAIL = N - _NS * RP
    mesh = plsc.VectorSubcoreMesh(
        core_axis_name="c", subcore_axis_name="s",
        num_cores=_NC, num_subcores=_NS)

    NCH = EW // K

    @functools.partial(
        pl.kernel,
        out_type=jax.ShapeDtypeStruct((_NC, N, H), jnp.float32),
        mesh=mesh,
        scratch_types=[
            pltpu.VMEM((NCH, K), jnp.int32),
            [pltpu.VMEM((K, H), jnp.float32)] * 2,
            pltpu.VMEM((ZB, H), jnp.float32),
            pltpu.VMEM_SHARED((N, H), jnp.float32),
            [pltpu.SemaphoreType.DMA] * 2,
        ],
    )
    def scatter_add(msg_hbm, dst3_hbm, agg_hbm, idxd, rows, zbuf, acc, sem):
        cid = lax.axis_index("c")
        sid = lax.axis_index("s")
        wid = sid * _NC + cid
        base = wid * EW
        pltpu.sync_copy(dst3_hbm.at[wid], idxd)
        for r in range(ZB):
            for cc in range(H // 16):
                zbuf[r, pl.ds(cc * 16, 16)] = jnp.zeros((16,), jnp.float32)
        for t in range(RP // ZB):
            pltpu.sync_copy(zbuf, acc.at[pl.ds(sid * RP + t * ZB, ZB)])

        @pl.when(sid == _NS - 1)
        def _zero_tail():
            for t in range(TAIL // ZB):
                pltpu.sync_copy(zbuf, acc.at[pl.ds(_NS * RP + t * ZB, ZB)])

        plsc.subcore_barrier()

        def start_load(c, p):
            pltpu.async_copy(msg_hbm.at[pl.ds(base + c * K, K)],
                             rows[p], sem[p])

        start_load(0, 0)
        start_load(1, 1)

        def do_chunk(c, p):
            pltpu.make_async_copy(
                msg_hbm.at[pl.ds(base, K)], rows[p], sem[p]).wait()
            pltpu.sync_copy(rows[p], acc.at[idxd.at[c]], add=True)

            @pl.when(c + 2 < NCH)
            def _next():
                start_load(c + 2, p)

        def body(t, carry):
            c0 = 2 * t
            do_chunk(c0, 0)

            @pl.when(c0 + 1 < NCH)
            def _odd():
                do_chunk(c0 + 1, 1)

            return carry

        lax.fori_loop(0, (NCH + 1) // 2, body, 0)
        plsc.subcore_barrier()
        pltpu.sync_copy(acc.at[pl.ds(sid * RP, RP)],
                        agg_hbm.at[cid, pl.ds(sid * RP, RP)])

        @pl.when(sid == _NS - 1)
        def _write_tail():
            pltpu.sync_copy(acc.at[pl.ds(_NS * RP, TAIL)],
                            agg_hbm.at[cid, pl.ds(_NS * RP, TAIL)])

    return scatter_add


def kernel(x, edge_index, edge_attr, batch, W_embed, b_embed,
           conv0_W1, conv0_b1, conv0_W2, conv0_b2,
           conv1_W1, conv1_b1, conv1_W2, conv1_b2,
           conv2_W1, conv2_b1, conv2_W2, conv2_b2,
           W_mlp1, b_mlp1, W_mlp2, b_mlp2):
    N, D = x.shape
    H = W_embed.shape[1]
    E = edge_index.shape[1]

    EA, KA = 192000, 120
    EB, KB = E - 192000, 80
    NCHA = (EA // _NW) // KA
    NCHB = (EB // _NW) // KB
    srcA3 = edge_index[0][:EA].reshape(_NW, NCHA, KA)
    dstA3 = edge_index[1][:EA].reshape(_NW, NCHA, KA)
    srcB3 = edge_index[0][EA:].reshape(_NW, NCHB, KB)
    dstB3 = edge_index[1][EA:].reshape(_NW, NCHB, KB)
    eaA = edge_attr[:EA]
    eaB = edge_attr[EA:]
    convs = [(conv0_W1, conv0_b1, conv0_W2, conv0_b2),
             (conv1_W1, conv1_b1, conv1_W2, conv1_b2),
             (conv2_W1, conv2_b1, conv2_W2, conv2_b2)]

    gatherA = _build_gather(N, H, EA, KA)
    scatterA = _build_scatter(N, H, EA, KA, 16)
    gatherB = _build_gather(N, H, EB, KB)
    scatterB = _build_scatter(N, H, EB, KB, 16)

    W1_0 = convs[0][0]
    h, a, b = _embed_prep(x, W_embed, b_embed.reshape(1, H),
                          W1_0[:H], W1_0[H:2 * H])

    out = None
    for l in range(3):
        W1, b1, W2, b2 = convs[l]
        sA = gatherA(a, b, srcA3, dstA3)
        sB = gatherB(a, b, srcB3, dstB3)
        msgA = _edge_mlp(sA, eaA, W1[2 * H:], b1.reshape(1, -1),
                         W2, b2.reshape(1, -1))
        msgB = _edge_mlp(sB, eaB, W1[2 * H:], b1.reshape(1, -1),
                         W2, b2.reshape(1, -1))
        aggA = scatterA(msgA, dstA3)
        aggB = scatterB(msgB, dstB3)
        if l < 2:
            W1n = convs[l + 1][0]
            h, a, b = _update_prep(h, aggA, aggB, W1n[:H], W1n[H:2 * H])
        else:
            out = _pool_head(h, aggA, aggB,
                             batch.reshape(N, 1), batch.reshape(1, N),
                             W_mlp1, b_mlp1.reshape(1, -1),
                             W_mlp2, b_mlp2.reshape(1, 1))
    return out
